# baseline (speedup 1.0000x reference)
import jax
import jax.numpy as jnp
from jax.experimental import pallas as pl

T, B, E, H, D = 200, 1024, 64, 2, 256
BB = 64


def _dnd_read_kernel(keys_ref, vals_ref, rpe_ref, q_ref, wt_ref, b_ref, out_ref):
    k = keys_ref[...]
    r = rpe_ref[...]
    q = q_ref[...]
    q0 = q[:, :E]
    q1 = q[:, E:]

    l0 = jnp.transpose(jnp.sum(k * q0[None, :, :], axis=-1)) * r
    l1 = jnp.transpose(jnp.sum(k * q1[None, :, :], axis=-1)) * r

    e0 = jnp.exp(l0 - jnp.max(l0, axis=-1, keepdims=True))
    w0 = e0 / jnp.sum(e0, axis=-1, keepdims=True)
    e1 = jnp.exp(l1 - jnp.max(l1, axis=-1, keepdims=True))
    w1 = e1 / jnp.sum(e1, axis=-1, keepdims=True)

    v = vals_ref[...]
    w0c = jnp.transpose(w0)[:, :, None]
    w1c = jnp.transpose(w1)[:, :, None]
    res0 = jnp.sum(w0c * v, axis=0)
    res1 = jnp.sum(w1c * v, axis=0)

    res = jnp.concatenate([res0, res1], axis=-1)
    out_ref[...] = (
        jnp.dot(res, wt_ref[...], preferred_element_type=jnp.float32)
        + b_ref[...]
    )


def kernel(keys, vals, rpe, query, W, b):
    rpe_bt = rpe.reshape(T, B).T
    q2 = query.reshape(B, H * E)
    wt = W.T
    b2 = b.reshape(1, D)

    grid = (B // BB,)
    return pl.pallas_call(
        _dnd_read_kernel,
        grid=grid,
        in_specs=[
            pl.BlockSpec((T, BB, E), lambda i: (0, i, 0)),
            pl.BlockSpec((T, BB, D), lambda i: (0, i, 0)),
            pl.BlockSpec((BB, T), lambda i: (i, 0)),
            pl.BlockSpec((BB, H * E), lambda i: (i, 0)),
            pl.BlockSpec((H * D, D), lambda i: (0, 0)),
            pl.BlockSpec((1, D), lambda i: (0, 0)),
        ],
        out_specs=pl.BlockSpec((BB, D), lambda i: (i, 0)),
        out_shape=jax.ShapeDtypeStruct((B, D), jnp.float32),
    )(keys, vals, rpe_bt, q2, wt, b2)
